# baseline (speedup 1.0000x reference)
import functools

import jax
import jax.numpy as jnp
from jax import lax
from jax.experimental import pallas as pl
from jax.experimental.pallas import tpu as pltpu
from jax.experimental.pallas import tpu_sc as plsc

Q = 8192
N = 8192
K = 3
BQ = 512
BN = 8192
NUM_WORKERS = 32
QW = Q // NUM_WORKERS
LANES = 16
W = 4

_INF = 3.0e38
_BIGF = 1.0e9


def _topk_body(qT_ref, r_ref, qsq_ref, rsq_ref, dist_ref, idx_ref):
    qsq = qsq_ref[0:1, :]
    gio_w = lax.broadcasted_iota(jnp.int32, (W, BQ), 0).astype(jnp.float32)

    def _dot(t):
        r2 = r_ref[pl.ds(t * BN, BN), :]
        return lax.dot_general(r2, qT_ref[:, :],
                               (((1,), (0,)), ((), ())),
                               preferred_element_type=jnp.float32)

    def chunk(t, carry):
        b1, b2, b3, i1, i2, i3 = carry
        acc = _dot(t)
        for g in range(BN // W):
            rsq_g = rsq_ref[pl.ds(t * BN + g * W, W), :]
            cur = jnp.maximum((qsq + rsq_g) - acc[g * W:(g + 1) * W, :], 0.0)
            gio = gio_w + (t * BN + g * W).astype(jnp.float32)
            c1 = cur < b1
            c2 = cur < b2
            c3 = cur < b3
            nb1 = jnp.minimum(cur, b1)
            nb2 = jnp.where(c2, jnp.maximum(cur, b1), b2)
            nb3 = jnp.where(c3, jnp.maximum(cur, b2), b3)
            ni1 = jnp.where(c1, gio, i1)
            ni2 = jnp.where(c2, jnp.where(c1, i1, gio), i2)
            ni3 = jnp.where(c3, jnp.where(c2, i2, gio), i3)
            b1, b2, b3, i1, i2, i3 = nb1, nb2, nb3, ni1, ni2, ni3
        return b1, b2, b3, i1, i2, i3

    full = lambda v: jnp.full((W, BQ), v, jnp.float32)
    b1, b2, b3, i1, i2, i3 = lax.fori_loop(
        0, N // BN, chunk,
        (full(_INF), full(_INF), full(_INF), full(-1.0), full(-2.0), full(-3.0)))

    cv = jnp.concatenate([b1, b2, b3], axis=0)
    ci = jnp.concatenate([i1, i2, i3], axis=0)
    ov, oi = [], []
    for _ in range(K):
        m = jnp.min(cv, axis=0, keepdims=True)
        sel = jnp.min(jnp.where(cv == m, ci, _BIGF), axis=0, keepdims=True)
        ov.append(m)
        oi.append(sel)
        cv = jnp.where(ci == sel, _INF, cv)
    dist_ref[:, :] = jnp.sqrt(jnp.concatenate(ov, axis=0))
    idx_ref[:, :] = jnp.concatenate(oi, axis=0).astype(jnp.int32)


def _topk_call(qT_bf16, r_bf16, qsq_row, rsq_col):
    return pl.pallas_call(
        _topk_body,
        grid=(Q // BQ,),
        in_specs=[
            pl.BlockSpec((3, BQ), lambda i: (0, i)),
            pl.BlockSpec((N, 3), lambda i: (0, 0)),
            pl.BlockSpec((1, BQ), lambda i: (0, i)),
            pl.BlockSpec((N, 1), lambda i: (0, 0)),
        ],
        out_specs=[
            pl.BlockSpec((K, BQ), lambda i: (0, i)),
            pl.BlockSpec((K, BQ), lambda i: (0, i)),
        ],
        out_shape=[
            jax.ShapeDtypeStruct((K, Q), jnp.float32),
            jax.ShapeDtypeStruct((K, Q), jnp.int32),
        ],
    )(qT_bf16, r_bf16, qsq_row, rsq_col)


def _sc_combine(dist_T, idx_T, flow_flat):
    mesh = plsc.VectorSubcoreMesh(core_axis_name="c", subcore_axis_name="s")

    @functools.partial(
        pl.kernel,
        mesh=mesh,
        compiler_params=pltpu.CompilerParams(needs_layout_passes=False),
        out_type=jax.ShapeDtypeStruct((K, Q), jnp.float32),
        scratch_types=[
            pltpu.VMEM((N * 3,), jnp.float32),
            pltpu.VMEM((K, QW), jnp.float32),
            pltpu.VMEM((K, QW), jnp.int32),
            pltpu.VMEM((K, QW), jnp.float32),
        ],
    )
    def body(dist_hbm, idx_hbm, flow_hbm, out_hbm, flow_v, dist_v, idx_v, out_v):
        wid = lax.axis_index("s") * 2 + lax.axis_index("c")
        base = wid * QW
        pltpu.sync_copy(flow_hbm, flow_v)
        pltpu.sync_copy(dist_hbm.at[:, pl.ds(base, QW)], dist_v)
        pltpu.sync_copy(idx_hbm.at[:, pl.ds(base, QW)], idx_v)
        for g in range(QW // LANES):
            sl = pl.ds(g * LANES, LANES)
            w0 = 1.0 / (dist_v[0, sl] + 1e-8)
            w1 = 1.0 / (dist_v[1, sl] + 1e-8)
            w2 = 1.0 / (dist_v[2, sl] + 1e-8)
            s = w0 + w1 + w2
            i0 = idx_v[0, sl] * 3
            i1 = idx_v[1, sl] * 3
            i2 = idx_v[2, sl] * 3
            for c in range(3):
                g0 = plsc.load_gather(flow_v, [i0 + c])
                g1 = plsc.load_gather(flow_v, [i1 + c])
                g2 = plsc.load_gather(flow_v, [i2 + c])
                out_v[c, sl] = (w0 * g0 + w1 * g1 + w2 * g2) / s
        pltpu.sync_copy(out_v, out_hbm.at[:, pl.ds(base, QW)])

    return body(dist_T, idx_T, flow_flat)


def kernel(query_points, ref_points, ref_flow, k):
    del k
    qsq = jnp.sum(query_points * query_points, axis=1)[None, :]
    rsq = jnp.sum(ref_points * ref_points, axis=1)[:, None]
    qT = query_points.T.astype(jnp.bfloat16)
    rb = (ref_points * 2.0).astype(jnp.bfloat16)
    dist_T, idx_T = _topk_call(qT, rb, qsq, rsq)
    out_T = _sc_combine(dist_T, idx_T, ref_flow.reshape(-1))
    return out_T.T


# speedup vs baseline: 1.8450x; 1.8450x over previous
import functools

import jax
import jax.numpy as jnp
from jax import lax
from jax.experimental import pallas as pl
from jax.experimental.pallas import tpu as pltpu
from jax.experimental.pallas import tpu_sc as plsc

Q = 8192
N = 8192
K = 3
BQ = 1024
BN = 4096
NUM_WORKERS = 32
QW = Q // NUM_WORKERS
LANES = 16
W = 8

_INF = 3.0e38
_BIGF = 1.0e9


def _topk_body(qT_ref, r_ref, qsq_ref, rsq_ref, dist_ref, idx_ref):
    qsq = qsq_ref[0:1, :]
    gio_w = lax.broadcasted_iota(jnp.int32, (W, BQ), 0).astype(jnp.float32)

    def _dot(t):
        r2 = r_ref[pl.ds(t * BN, BN), :]
        return lax.dot_general(r2, qT_ref[:, :],
                               (((1,), (0,)), ((), ())),
                               preferred_element_type=jnp.float32)

    def chunk(t, carry):
        b1, b2, b3, i1, i2, i3 = carry
        acc = _dot(t)
        for g in range(BN // W):
            rsq_g = rsq_ref[pl.ds(t * BN + g * W, W), :]
            cur = jnp.maximum((qsq + rsq_g) - acc[g * W:(g + 1) * W, :], 0.0)
            gio = gio_w + (t * BN + g * W).astype(jnp.float32)
            c1 = cur < b1
            c2 = cur < b2
            c3 = cur < b3
            nb1 = jnp.minimum(cur, b1)
            nb2 = jnp.where(c2, jnp.maximum(cur, b1), b2)
            nb3 = jnp.where(c3, jnp.maximum(cur, b2), b3)
            ni1 = jnp.where(c1, gio, i1)
            ni2 = jnp.where(c2, jnp.where(c1, i1, gio), i2)
            ni3 = jnp.where(c3, jnp.where(c2, i2, gio), i3)
            b1, b2, b3, i1, i2, i3 = nb1, nb2, nb3, ni1, ni2, ni3
        return b1, b2, b3, i1, i2, i3

    full = lambda v: jnp.full((W, BQ), v, jnp.float32)
    b1, b2, b3, i1, i2, i3 = lax.fori_loop(
        0, N // BN, chunk,
        (full(_INF), full(_INF), full(_INF), full(-1.0), full(-2.0), full(-3.0)))

    cv = jnp.concatenate([b1, b2, b3], axis=0)
    ci = jnp.concatenate([i1, i2, i3], axis=0)
    ov, oi = [], []
    for _ in range(K):
        m = jnp.min(cv, axis=0, keepdims=True)
        sel = jnp.min(jnp.where(cv == m, ci, _BIGF), axis=0, keepdims=True)
        ov.append(m)
        oi.append(sel)
        cv = jnp.where(ci == sel, _INF, cv)
    dist_ref[:, :] = jnp.sqrt(jnp.concatenate(ov, axis=0))
    idx_ref[:, :] = jnp.concatenate(oi, axis=0).astype(jnp.int32)


def _topk_call(qT_bf16, r_bf16, qsq_row, rsq_col):
    return pl.pallas_call(
        _topk_body,
        grid=(Q // BQ,),
        in_specs=[
            pl.BlockSpec((3, BQ), lambda i: (0, i)),
            pl.BlockSpec((N, 3), lambda i: (0, 0)),
            pl.BlockSpec((1, BQ), lambda i: (0, i)),
            pl.BlockSpec((N, 1), lambda i: (0, 0)),
        ],
        out_specs=[
            pl.BlockSpec((K, BQ), lambda i: (0, i)),
            pl.BlockSpec((K, BQ), lambda i: (0, i)),
        ],
        out_shape=[
            jax.ShapeDtypeStruct((K, Q), jnp.float32),
            jax.ShapeDtypeStruct((K, Q), jnp.int32),
        ],
    )(qT_bf16, r_bf16, qsq_row, rsq_col)


def _sc_combine(dist_T, idx_T, flow_flat):
    mesh = plsc.VectorSubcoreMesh(core_axis_name="c", subcore_axis_name="s")

    @functools.partial(
        pl.kernel,
        mesh=mesh,
        compiler_params=pltpu.CompilerParams(needs_layout_passes=False),
        out_type=jax.ShapeDtypeStruct((K, Q), jnp.float32),
        scratch_types=[
            pltpu.VMEM((N * 3,), jnp.float32),
            pltpu.VMEM((K, QW), jnp.float32),
            pltpu.VMEM((K, QW), jnp.int32),
            pltpu.VMEM((K, QW), jnp.float32),
        ],
    )
    def body(dist_hbm, idx_hbm, flow_hbm, out_hbm, flow_v, dist_v, idx_v, out_v):
        wid = lax.axis_index("s") * 2 + lax.axis_index("c")
        base = wid * QW
        pltpu.sync_copy(flow_hbm, flow_v)
        pltpu.sync_copy(dist_hbm.at[:, pl.ds(base, QW)], dist_v)
        pltpu.sync_copy(idx_hbm.at[:, pl.ds(base, QW)], idx_v)
        for g in range(QW // LANES):
            sl = pl.ds(g * LANES, LANES)
            w0 = 1.0 / (dist_v[0, sl] + 1e-8)
            w1 = 1.0 / (dist_v[1, sl] + 1e-8)
            w2 = 1.0 / (dist_v[2, sl] + 1e-8)
            s = w0 + w1 + w2
            i0 = idx_v[0, sl] * 3
            i1 = idx_v[1, sl] * 3
            i2 = idx_v[2, sl] * 3
            for c in range(3):
                g0 = plsc.load_gather(flow_v, [i0 + c])
                g1 = plsc.load_gather(flow_v, [i1 + c])
                g2 = plsc.load_gather(flow_v, [i2 + c])
                out_v[c, sl] = (w0 * g0 + w1 * g1 + w2 * g2) / s
        pltpu.sync_copy(out_v, out_hbm.at[:, pl.ds(base, QW)])

    return body(dist_T, idx_T, flow_flat)


def kernel(query_points, ref_points, ref_flow, k):
    del k
    qsq = jnp.sum(query_points * query_points, axis=1)[None, :]
    rsq = jnp.sum(ref_points * ref_points, axis=1)[:, None]
    qT = query_points.T.astype(jnp.bfloat16)
    rb = (ref_points * 2.0).astype(jnp.bfloat16)
    dist_T, idx_T = _topk_call(qT, rb, qsq, rsq)
    out_T = _sc_combine(dist_T, idx_T, ref_flow.reshape(-1))
    return out_T.T
